# Initial kernel scaffold; baseline (speedup 1.0000x reference)
import jax
import jax.numpy as jnp
from jax.experimental import pallas as pl


def kernel(x, Wg, bg, W1, b1, W2, b2):
    raise NotImplementedError("write your pallas kernel here")


# speedup vs baseline: 1.2755x; 1.2755x over previous
import functools

import jax
import jax.numpy as jnp
from jax import lax
from jax.experimental import pallas as pl
from jax.experimental.pallas import tpu as pltpu
from jax.experimental.pallas import tpu_sc as plsc

_BS = 256
_NW = 32


def _sc_gather(row_token, x_pad, R, D):
    rows_per_w = R // _NW
    CH = 64
    nch = rows_per_w // CH
    mesh = plsc.VectorSubcoreMesh(core_axis_name="c", subcore_axis_name="s")

    @functools.partial(
        pl.kernel,
        mesh=mesh,
        out_type=jax.ShapeDtypeStruct((R, D), jnp.float32),
        scratch_types=[
            pltpu.VMEM((CH,), jnp.int32),
            pltpu.VMEM((CH, D), jnp.float32),
            pltpu.SemaphoreType.DMA,
        ],
    )
    def k(tok_hbm, x_hbm, out_hbm, idx_v, rows_v, sem):
        wid = lax.axis_index("s") * 2 + lax.axis_index("c")
        base = wid * rows_per_w
        for c in range(nch):
            off = base + c * CH
            pltpu.sync_copy(tok_hbm.at[pl.ds(off, CH)], idx_v)
            pltpu.async_copy(x_hbm.at[idx_v], rows_v, sem).wait()
            pltpu.sync_copy(rows_v, out_hbm.at[pl.ds(off, CH)])

    return k(row_token, x_pad)


def _sc_combine(p0, p1, ys, T, D):
    toks_per_w = T // _NW
    CH = 32
    nch = toks_per_w // CH
    ncol = D // 16
    mesh = plsc.VectorSubcoreMesh(core_axis_name="c", subcore_axis_name="s")

    @functools.partial(
        pl.kernel,
        mesh=mesh,
        out_type=jax.ShapeDtypeStruct((T, D), jnp.float32),
        scratch_types=[
            pltpu.VMEM((CH,), jnp.int32),
            pltpu.VMEM((CH,), jnp.int32),
            pltpu.VMEM((CH, D), jnp.float32),
            pltpu.VMEM((CH, D), jnp.float32),
            pltpu.SemaphoreType.DMA,
        ],
    )
    def k(p0_hbm, p1_hbm, ys_hbm, out_hbm, i0_v, i1_v, r0_v, r1_v, sem):
        wid = lax.axis_index("s") * 2 + lax.axis_index("c")
        base = wid * toks_per_w
        for c in range(nch):
            off = base + c * CH
            pltpu.sync_copy(p0_hbm.at[pl.ds(off, CH)], i0_v)
            pltpu.sync_copy(p1_hbm.at[pl.ds(off, CH)], i1_v)
            cp0 = pltpu.async_copy(ys_hbm.at[i0_v], r0_v, sem)
            cp1 = pltpu.async_copy(ys_hbm.at[i1_v], r1_v, sem)
            cp0.wait()
            cp1.wait()

            def row_body(r, _):
                def col_body(j, _):
                    sl = pl.ds(j * 16, 16)
                    r0_v[r, sl] = r0_v[r, sl] + r1_v[r, sl]
                    return 0

                return lax.fori_loop(0, ncol, col_body, 0)

            lax.fori_loop(0, CH, row_body, 0)
            pltpu.sync_copy(r0_v, out_hbm.at[pl.ds(off, CH)])

    return k(p0, p1, ys)


def _gmm_body(be_ref, xs_ref, w1_ref, b1_ref, w2_ref, b2_ref, rw_ref, out_ref):
    xb = xs_ref[...].astype(jnp.bfloat16)
    h = jnp.dot(xb, w1_ref[0], preferred_element_type=jnp.float32)
    h = jnp.maximum(h + b1_ref[0], 0.0).astype(jnp.bfloat16)
    y = jnp.dot(h, w2_ref[0], preferred_element_type=jnp.float32)
    out_ref[...] = (y + b2_ref[0]) * rw_ref[:, :1]


def _gmm(block_expert, xs, W1, b1, W2, b2, rw, nblk, R, D, H):
    grid_spec = pltpu.PrefetchScalarGridSpec(
        num_scalar_prefetch=1,
        grid=(nblk,),
        in_specs=[
            pl.BlockSpec((_BS, D), lambda i, be: (i, 0)),
            pl.BlockSpec((1, D, H), lambda i, be: (be[i], 0, 0)),
            pl.BlockSpec((1, 1, H), lambda i, be: (be[i], 0, 0)),
            pl.BlockSpec((1, H, D), lambda i, be: (be[i], 0, 0)),
            pl.BlockSpec((1, 1, D), lambda i, be: (be[i], 0, 0)),
            pl.BlockSpec((_BS, 128), lambda i, be: (i, 0)),
        ],
        out_specs=pl.BlockSpec((_BS, D), lambda i, be: (i, 0)),
    )
    return pl.pallas_call(
        _gmm_body,
        grid_spec=grid_spec,
        out_shape=jax.ShapeDtypeStruct((R, D), jnp.float32),
        compiler_params=pltpu.CompilerParams(
            dimension_semantics=("arbitrary",),
            vmem_limit_bytes=100 * 1024 * 1024,
        ),
    )(block_expert, xs, W1, b1, W2, b2, rw)


def kernel(x, Wg, bg, W1, b1, W2, b2):
    B, S, D = x.shape
    E = Wg.shape[1]
    H = W1.shape[2]
    K = 2
    T = B * S
    nblk = (K * T) // _BS + E
    R = nblk * _BS

    gate_scores = jnp.einsum('bsd,de->bse', x, Wg) + bg
    topk_scores, topk_idx = jax.lax.top_k(gate_scores, K)
    topk_w = jax.nn.softmax(topk_scores, axis=-1)

    e_flat = topk_idx.reshape(T * K).astype(jnp.int32)
    w_flat = topk_w.reshape(T * K).astype(jnp.float32)
    tok_of_a = (jnp.arange(T * K, dtype=jnp.int32) // K)

    onehot = (e_flat[:, None] == jnp.arange(E, dtype=jnp.int32)[None, :])
    csum = jnp.cumsum(onehot.astype(jnp.int32), axis=0)
    counts = csum[-1]
    rank = jnp.take_along_axis(csum, e_flat[:, None], axis=1)[:, 0] - 1
    blocks_per_e = (counts + _BS - 1) // _BS
    off_blocks = jnp.concatenate(
        [jnp.zeros((1,), jnp.int32), jnp.cumsum(blocks_per_e).astype(jnp.int32)])
    pos = off_blocks[e_flat] * _BS + rank

    row_token = jnp.full((R,), T, jnp.int32).at[pos].set(tok_of_a)
    row_w = jnp.zeros((R,), jnp.float32).at[pos].set(w_flat)
    block_expert = jnp.minimum(
        jnp.searchsorted(off_blocks[1:], jnp.arange(nblk, dtype=jnp.int32),
                         side='right'),
        E - 1).astype(jnp.int32)
    pos2 = pos.reshape(T, K)

    x_pad = jnp.concatenate([x.reshape(T, D),
                             jnp.zeros((1, D), jnp.float32)], axis=0)
    xs = _sc_gather(row_token, x_pad, R, D)

    rw2 = jnp.broadcast_to(row_w[:, None], (R, 128))
    ys = _gmm(block_expert, xs,
              W1.astype(jnp.bfloat16), b1.reshape(E, 1, H),
              W2.astype(jnp.bfloat16), b2.reshape(E, 1, D),
              rw2, nblk, R, D, H)

    out = _sc_combine(pos2[:, 0], pos2[:, 1], ys, T, D)
    return out.reshape(B, S, D)
